# baseline (speedup 1.0000x reference)
import functools

import jax
import jax.numpy as jnp
from jax import lax
from jax.experimental import pallas as pl
from jax.experimental.pallas import tpu as pltpu
from jax.experimental.pallas import tpu_sc as plsc

N_CLS = 4096
N_CTX = 4
CTX_DIM = 512
CTX_LEN = 77
N_SUF = CTX_LEN - N_CTX - 1
PAD_LEN = 80
SUF_COL = 8
NBUF = 2

_info = plsc.get_sparse_core_info()
_NC = _info.num_cores
_NS = _info.num_subcores
_NW = _NC * _NS
_CPT = N_CLS // _NW
_NSTEP = _CPT // NBUF


def _make_sc_call():
  mesh = plsc.VectorSubcoreMesh(core_axis_name="c", subcore_axis_name="s")

  @functools.partial(
      pl.kernel,
      mesh=mesh,
      compiler_params=pltpu.CompilerParams(use_tc_tiling_on_sc=False),
      out_type=jax.ShapeDtypeStruct((N_CLS, CTX_LEN, CTX_DIM), jnp.float32),
      scratch_types=[
          pltpu.VMEM((_CPT, PAD_LEN), jnp.int32),
          pltpu.VMEM((NBUF, PAD_LEN, CTX_DIM), jnp.float32),
          pltpu.VMEM((N_CTX, CTX_DIM), jnp.float32),
      ] + [pltpu.SemaphoreType.DMA] * (2 * NBUF),
  )
  def sc_kernel(tp_hbm, ctx_hbm, table_hbm, out_hbm, idx_v, rows_v, ctx_v,
                *sems):
    gsem = sems[:NBUF]
    wsem = sems[NBUF:]
    wid = lax.axis_index("s") * _NC + lax.axis_index("c")
    base = wid * _CPT

    pltpu.sync_copy(tp_hbm.at[pl.ds(base, _CPT)], idx_v)
    pltpu.sync_copy(ctx_hbm, ctx_v)

    def gather(il, b):
      return pltpu.make_async_copy(table_hbm.at[idx_v.at[il]], rows_v.at[b],
                                   gsem[b])

    def writebacks(il, b):
      gc = base + il
      return (
          pltpu.make_async_copy(rows_v.at[b, pl.ds(0, 1)],
                                out_hbm.at[gc, pl.ds(0, 1)], wsem[b]),
          pltpu.make_async_copy(ctx_v, out_hbm.at[gc, pl.ds(1, N_CTX)],
                                wsem[b]),
          pltpu.make_async_copy(rows_v.at[b, pl.ds(SUF_COL, N_SUF)],
                                out_hbm.at[gc, pl.ds(1 + N_CTX, N_SUF)],
                                wsem[b]),
      )

    def finish_class(il, b):
      gather(il, b).wait()
      for d in writebacks(il, b):
        d.start()

    def drain_writes(il, b):
      for d in writebacks(il, b):
        d.wait()

    for b in range(NBUF):
      gather(b, b).start()

    def body(g, carry):
      il0 = g * NBUF
      for b in range(NBUF):
        il = il0 + b
        finish_class(il, b)
        drain_writes(il, b)
        gather(il + NBUF, b).start()
      return carry

    lax.fori_loop(0, _NSTEP - 1, body, 0)

    il0 = (_NSTEP - 1) * NBUF
    for b in range(NBUF):
      il = il0 + b
      finish_class(il, b)
      drain_writes(il, b)

  return sc_kernel


_sc_call = _make_sc_call()


def kernel(tokenized_prompts, ctx, token_embedding):
  tp_idx = jnp.concatenate(
      [
          tokenized_prompts[:, :1],
          jnp.zeros((N_CLS, SUF_COL - 1), jnp.int32),
          tokenized_prompts[:, 1:1 + N_SUF],
      ],
      axis=1,
  )
  prompts = _sc_call(tp_idx, ctx, token_embedding)
  return (prompts, tokenized_prompts)


# speedup vs baseline: 1.2921x; 1.2921x over previous
import functools

import jax
import jax.numpy as jnp
from jax import lax
from jax.experimental import pallas as pl
from jax.experimental.pallas import tpu as pltpu
from jax.experimental.pallas import tpu_sc as plsc

N_CLS = 4096
N_CTX = 4
CTX_DIM = 512
CTX_LEN = 77
N_SUF = CTX_LEN - N_CTX - 1
VOCAB = 49408
PAD_LEN = 80

_info = plsc.get_sparse_core_info()
_NC = _info.num_cores
_NS = _info.num_subcores
_NW = _NC * _NS
_CPT = N_CLS // _NW


def _make_sc_call():
  mesh = plsc.VectorSubcoreMesh(core_axis_name="c", subcore_axis_name="s")

  @functools.partial(
      pl.kernel,
      mesh=mesh,
      out_type=jax.ShapeDtypeStruct((N_CLS, CTX_LEN, CTX_DIM), jnp.float32),
      compiler_params=pltpu.CompilerParams(use_tc_tiling_on_sc=False),
      scratch_types=[
          pltpu.VMEM((PAD_LEN,), jnp.int32),
          pltpu.VMEM((PAD_LEN, CTX_DIM), jnp.float32),
          pltpu.SemaphoreType.DMA,
      ],
  )
  def sc_kernel(idx_hbm, table_hbm, out_hbm, idx_v, rows_v, sem):
    wid = lax.axis_index("s") * _NC + lax.axis_index("c")
    base = wid * _CPT

    def body(i, carry):
      gc = base + i
      pltpu.sync_copy(idx_hbm.at[pl.ds(gc * PAD_LEN, PAD_LEN)], idx_v)
      pltpu.async_copy(table_hbm.at[idx_v], rows_v, sem).wait()
      pltpu.sync_copy(rows_v.at[pl.ds(0, CTX_LEN)], out_hbm.at[gc])
      return carry

    lax.fori_loop(0, _CPT, body, 0)

  return sc_kernel


_sc_call = _make_sc_call()


def kernel(tokenized_prompts, ctx, token_embedding):
  table_ext = jnp.concatenate([token_embedding, ctx], axis=0)
  ctx_ids = jnp.broadcast_to(
      jnp.arange(VOCAB, VOCAB + N_CTX, dtype=jnp.int32)[None, :],
      (N_CLS, N_CTX))
  idx = jnp.concatenate(
      [
          tokenized_prompts[:, :1],
          ctx_ids,
          tokenized_prompts[:, 1:1 + N_SUF],
          jnp.zeros((N_CLS, PAD_LEN - CTX_LEN), jnp.int32),
      ],
      axis=1,
  ).reshape(-1)
  prompts = _sc_call(idx, table_ext)
  return (prompts, tokenized_prompts)


# speedup vs baseline: 1.7730x; 1.3722x over previous
import functools

import jax
import jax.numpy as jnp
from jax import lax
from jax.experimental import pallas as pl
from jax.experimental.pallas import tpu as pltpu
from jax.experimental.pallas import tpu_sc as plsc

N_CLS = 4096
N_CTX = 4
CTX_DIM = 512
CTX_LEN = 77
N_SUF = CTX_LEN - N_CTX - 1
VOCAB = 49408
ROWS = N_CLS * CTX_LEN

B = 64
S = 3
A = 2

_info = plsc.get_sparse_core_info()
_NC = _info.num_cores
_NS = _info.num_subcores
_NW = _NC * _NS
_RPT = ROWS // _NW
_U = _RPT // B
assert ROWS % _NW == 0 and _RPT % B == 0


def _make_sc_call():
  mesh = plsc.VectorSubcoreMesh(core_axis_name="c", subcore_axis_name="s")

  @functools.partial(
      pl.kernel,
      mesh=mesh,
      compiler_params=pltpu.CompilerParams(use_tc_tiling_on_sc=False),
      out_type=jax.ShapeDtypeStruct((ROWS, CTX_DIM), jnp.float32),
      scratch_types=[pltpu.VMEM((B,), jnp.int32) for _ in range(S)]
      + [pltpu.VMEM((B, CTX_DIM), jnp.float32) for _ in range(S)]
      + [pltpu.SemaphoreType.DMA] * (3 * S),
  )
  def sc_kernel(idx_hbm, table_hbm, out_hbm, *rest):
    idxs = rest[:S]
    rows = rest[S:2 * S]
    isem = rest[2 * S:3 * S]
    gsem = rest[3 * S:4 * S]
    wsem = rest[4 * S:5 * S]
    wid = lax.axis_index("s") * _NC + lax.axis_index("c")
    base = wid * _RPT

    def i_copy(u, s):
      return pltpu.make_async_copy(idx_hbm.at[pl.ds(base + u * B, B)],
                                   idxs[s], isem[s])

    def g_copy(s):
      return pltpu.make_async_copy(table_hbm.at[idxs[s]], rows[s], gsem[s])

    def w_copy(u, s):
      return pltpu.make_async_copy(rows[s],
                                   out_hbm.at[pl.ds(base + u * B, B)],
                                   wsem[s])

    def tick(u, k, drain_w, ahead, inext):
      sa = (k + A) % S
      if drain_w:
        w_copy(u - (S - A), sa).wait()
      if ahead:
        i_copy(u + A, sa).wait()
        g_copy(sa).start()
      g_copy(k).wait()
      w_copy(u, k).start()
      if inext:
        i_copy(u + S, k).start()

    for s in range(S):
      i_copy(s, s).start()
    for k in range(A):
      i_copy(k, k).wait()
      g_copy(k).start()

    front = S - A
    steady_len = ((_U - 2 * S + A) // S) * S
    steady_end = front + steady_len

    for u in range(front):
      tick(u, u % S, drain_w=False, ahead=True, inext=True)

    def body(gi, carry):
      u0 = front + gi * S
      for k2 in range(S):
        tick(u0 + k2, (front + k2) % S, drain_w=True, ahead=True, inext=True)
      return carry

    lax.fori_loop(0, steady_len // S, body, 0)

    for u in range(steady_end, _U):
      tick(u, u % S, drain_w=True, ahead=(u + A < _U), inext=(u + S < _U))
    for u in range(_U - (S - A), _U):
      w_copy(u, u % S).wait()

  return sc_kernel


_sc_call = _make_sc_call()


def kernel(tokenized_prompts, ctx, token_embedding):
  table_ext = jnp.concatenate([token_embedding, ctx], axis=0)
  ctx_ids = jnp.broadcast_to(
      jnp.arange(VOCAB, VOCAB + N_CTX, dtype=jnp.int32)[None, :],
      (N_CLS, N_CTX))
  idx = jnp.concatenate(
      [
          tokenized_prompts[:, :1],
          ctx_ids,
          tokenized_prompts[:, 1:1 + N_SUF],
      ],
      axis=1,
  ).reshape(-1)
  prompts = _sc_call(idx, table_ext)
  return (prompts.reshape(N_CLS, CTX_LEN, CTX_DIM), tokenized_prompts)
